# baseline (speedup 1.0000x reference)
import dataclasses
import functools
import math

import jax
import jax.numpy as jnp
from jax import lax
from jax.experimental import pallas as pl
from jax.experimental.pallas import tpu as pltpu
from jax.experimental.pallas import tpu_sc as plsc

B = 4
S = 2048
D = 768
N = B * S
NC, NS = 2, 16
NW = NC * NS
SPAN = S // NW
G = 32
HPS = SPAN // G
NCHUNK = B * HPS
LANES = 16
KSCALE = math.sqrt(D)
NOSEP = 2 * S


def _prep_body(sep_ref, x_ref, pos_ref, seg_ref, out_ref, dseg_ref, fs_ref):
    out_ref[...] = (pos_ref[...] + seg_ref[0][None, :]) * KSCALE
    dseg_ref[...] = jnp.broadcast_to(
        (seg_ref[1] - seg_ref[0])[None, :] * KSCALE, (8, D))
    pos_idx = lax.broadcasted_iota(jnp.int32, (B, S), 1)
    cand = jnp.where(x_ref[...] == sep_ref[0], pos_idx, NOSEP)
    first = jnp.min(cand, axis=1, keepdims=True)
    fs_ref[...] = jnp.concatenate(
        [jnp.broadcast_to(first, (B, 128)),
         jnp.full((8 - B, 128), NOSEP, jnp.int32)], axis=0)


def _make_prep(sep_arr, x, pos_table, seg_table):
    return pl.pallas_call(
        _prep_body,
        in_specs=[
            pl.BlockSpec(memory_space=pltpu.SMEM),
            pl.BlockSpec((B, S), lambda: (0, 0)),
            pl.BlockSpec((S, D), lambda: (0, 0)),
            pl.BlockSpec((2, D), lambda: (0, 0)),
        ],
        out_specs=[
            pl.BlockSpec((S, D), lambda: (0, 0)),
            pl.BlockSpec((8, D), lambda: (0, 0)),
            pl.BlockSpec((8, 128), lambda: (0, 0)),
        ],
        out_shape=[
            jax.ShapeDtypeStruct((S, D), jnp.float32),
            jax.ShapeDtypeStruct((8, D), jnp.float32),
            jax.ShapeDtypeStruct((8, 128), jnp.int32),
        ],
    )(sep_arr, x, pos_table, seg_table)


_SC_CP = pltpu.CompilerParams()
if "needs_layout_passes" in pltpu.CompilerParams.__dataclass_fields__:
    _SC_CP = dataclasses.replace(_SC_CP, needs_layout_passes=False)


@functools.partial(
    pl.kernel,
    out_type=jax.ShapeDtypeStruct((N, D), jnp.float32),
    compiler_params=_SC_CP,
    mesh=plsc.VectorSubcoreMesh(core_axis_name="c", subcore_axis_name="s"),
    scratch_types=[
        pltpu.VMEM((B, SPAN), jnp.int32),
        pltpu.VMEM((B, LANES), jnp.int32),
        pltpu.VMEM((D,), jnp.float32),
        pltpu.VMEM((SPAN, D), jnp.float32),
        pltpu.VMEM((G, D), jnp.float32),
        pltpu.VMEM((G, D), jnp.float32),
        pltpu.VMEM((G, D), jnp.float32),
        pltpu.SemaphoreType.DMA,
        pltpu.SemaphoreType.DMA,
        pltpu.SemaphoreType.DMA,
        pltpu.SemaphoreType.DMA,
        pltpu.SemaphoreType.DMA,
        pltpu.SemaphoreType.DMA,
        pltpu.SemaphoreType.DMA,
        pltpu.SemaphoreType.DMA,
    ],
)
def _sc_lookup(fs_hbm, x_hbm, token_hbm, posk0_hbm, dseg_hbm, out_hbm,
               idx_v, fs_v, dseg_v, p_v, t0, t1, t2,
               saux, spn, st0, st1, st2, sw0, sw1, sw2):
    cid = lax.axis_index("c")
    sid = lax.axis_index("s")
    wid = sid * NC + cid
    span0 = wid * SPAN

    cps = [
        pltpu.async_copy(x_hbm.at[b, pl.ds(span0, SPAN)], idx_v.at[b], saux)
        for b in range(B)
    ]
    cp_p = pltpu.async_copy(posk0_hbm.at[pl.ds(span0, SPAN)], p_v, spn)
    cp_ds = pltpu.async_copy(dseg_hbm.at[0], dseg_v, saux)
    cps_fs = [
        pltpu.async_copy(fs_hbm.at[b, pl.ds(0, LANES)], fs_v.at[b], saux)
        for b in range(B)
    ]
    for cp in cps:
        cp.wait()

    slots = ((t0, st0, sw0), (t1, st1, sw1), (t2, st2, sw2))
    NSLOT = len(slots)

    def issue(c, slot):
        tb, st, _ = slot
        b, h = c // HPS, c % HPS
        pltpu.async_copy(token_hbm.at[idx_v.at[b, pl.ds(h * G, G)]], tb, st)

    def wait_gather(slot):
        tb, st, _ = slot
        pltpu.make_async_copy(token_hbm.at[pl.ds(0, G)], tb, st).wait()

    def wait_wb(slot):
        tb, _, sw = slot
        pltpu.make_async_copy(tb, out_hbm.at[pl.ds(0, G)], sw).wait()

    issue(0, slots[0])
    issue(1, slots[1])

    cp_ds.wait()
    for cp in cps_fs:
        cp.wait()
    p_first = [jnp.min(fs_v[b, pl.ds(0, LANES)]) for b in range(B)]
    cp_p.wait()

    NTRIP = -(-NCHUNK // NSLOT)

    @pl.loop(0, NTRIP)
    def _(it):
        for k_, slot in enumerate(slots):
            c = NSLOT * it + k_
            tb = slot[0]

            @pl.when(c < NCHUNK)
            def _(c=c, slot=slot, tb=tb):
                b, h = c // HPS, c % HPS
                pf = p_first[B - 1]
                for b2 in range(B - 1):
                    pf = jnp.where(b == b2, p_first[b2], pf)
                wait_gather(slot)
                jcut = jnp.clip(pf - (span0 + h * G), 0, G)
                poff = h * G

                @plsc.parallel_loop(0, jcut)
                def _(j):
                    for c2 in range(D // LANES):
                        sl = pl.ds(c2 * LANES, LANES)
                        tb[j, sl] = tb[j, sl] * KSCALE + p_v[poff + j, sl]

                @plsc.parallel_loop(jcut, G)
                def _(j):
                    for c2 in range(D // LANES):
                        sl = pl.ds(c2 * LANES, LANES)
                        tb[j, sl] = (tb[j, sl] * KSCALE
                                     + p_v[poff + j, sl] + dseg_v[sl])

                pltpu.async_copy(
                    tb, out_hbm.at[pl.ds(b * S + span0 + h * G, G)],
                    slot[2])

                nslot = slots[(k_ + 2) % NSLOT]

                @pl.when(c + 2 < NCHUNK)
                def _(c=c, nslot=nslot):
                    @pl.when(c >= 1)
                    def _():
                        wait_wb(nslot)

                    issue(c + 2, nslot)

    for slot in slots:
        wait_wb(slot)


def kernel(x, sep_token, token_table, pos_table, seg_table):
    sep_arr = jnp.asarray(sep_token, jnp.int32).reshape(1)
    posk0, dsegk, firstsep = _make_prep(sep_arr, x, pos_table, seg_table)
    out = _sc_lookup(firstsep, x, token_table, posk0, dsegk)
    return out.reshape(B, S, D)
